# Initial kernel scaffold; baseline (speedup 1.0000x reference)
import jax
import jax.numpy as jnp
from jax.experimental import pallas as pl


def kernel(x, edge_index, edge_attr, pos, params):
    raise NotImplementedError("write your pallas kernel here")


# speedup vs baseline: 1.0265x; 1.0265x over previous
import functools

import jax
import jax.numpy as jnp
import numpy as np
from jax.experimental import pallas as pl
from jax.experimental.pallas import tpu as pltpu

N_NODES = 10000
N_EDGES = 160000
K_SIZE = 25


_ROWS = 128
_NPAD = 10240


def _cdist_top6_body(posT_ref, sq_ref, pos_ref, out_ref):
    r = pos_ref[...]
    sq_r = jnp.sum(r * r, axis=1, keepdims=True)
    d2 = sq_r + sq_ref[0:1, :] - 2.0 * jnp.dot(
        r, posT_ref[...], preferred_element_type=jnp.float32)
    dist = jnp.sqrt(jnp.maximum(d2, 0.0) + 1e-12)
    col = jax.lax.broadcasted_iota(jnp.int32, dist.shape, 1)
    big = jnp.float32(3.0e38)
    dist = jnp.where(col < N_NODES, dist, big)
    total = jnp.zeros((_ROWS, 1), jnp.float32)
    min6 = jnp.zeros((_ROWS, 1), jnp.float32)
    for it in range(6):
        m = jnp.min(dist, axis=1, keepdims=True)
        cand = jnp.where(dist <= m, col, jnp.int32(2**30))
        mi = jnp.min(cand, axis=1, keepdims=True)
        dist = jnp.where(col == mi, big, dist)
        total = total + m
        if it == 0:
            min6 = m
    res = jnp.concatenate([total, min6] + [jnp.zeros((_ROWS, 1), jnp.float32)] * 126,
                          axis=1)
    out_ref[...] = res


def _edge_feat_pallas(pos, src):
    posp = jnp.zeros((_NPAD, 8), jnp.float32).at[:N_NODES, :3].set(pos)
    posT = jnp.zeros((8, _NPAD), jnp.float32).at[:3, :N_NODES].set(pos.T)
    sq = jnp.sum(posT * posT, axis=0, keepdims=True)
    sq8 = jnp.broadcast_to(sq, (8, _NPAD))
    grid = _NPAD // _ROWS
    out = pl.pallas_call(
        _cdist_top6_body,
        grid=(grid,),
        in_specs=[
            pl.BlockSpec((8, _NPAD), lambda i: (0, 0)),
            pl.BlockSpec((8, _NPAD), lambda i: (0, 0)),
            pl.BlockSpec((_ROWS, 8), lambda i: (i, 0)),
        ],
        out_specs=pl.BlockSpec((_ROWS, 128), lambda i: (i, 0)),
        out_shape=jax.ShapeDtypeStruct((_NPAD, 128), jnp.float32),
    )(posT, sq8, posp)
    sum6 = out[:N_NODES, 0]
    min6 = out[:N_NODES, 1]
    avg = ((sum6 - min6) / 5.0)[:, None]
    deg = jnp.zeros((N_NODES,), jnp.float32).at[src].add(1.0)[:, None]
    dens = 1.0 / (avg + 1e-6)
    return jnp.concatenate([avg, deg, dens], axis=1)


def _linear(x, p):
    return x @ p["w"].T + p["b"]


def _bn(x, p):
    mu = x.mean(axis=0)
    var = x.var(axis=0)
    return (x - mu) / jnp.sqrt(var + 1e-5) * p["g"] + p["beta"]


def _ln(x, p):
    mu = x.mean(axis=-1, keepdims=True)
    var = x.var(axis=-1, keepdims=True)
    return (x - mu) / jnp.sqrt(var + 1e-5) * p["g"] + p["beta"]


def _mlp(x, p):
    return _linear(jax.nn.gelu(_linear(x, p["l0"]), approximate=False), p["l1"])


def _spline_basis(pseudo):
    K = K_SIZE
    v = pseudo * (K - 1)
    i0f = jnp.clip(jnp.floor(v), 0.0, K - 2)
    frac = v - i0f
    i0 = i0f.astype(jnp.int32)
    bs, ws = [], []
    for s in range(8):
        bits = [(s >> d) & 1 for d in range(3)]
        b = jnp.ones((pseudo.shape[0],), pseudo.dtype)
        widx = jnp.zeros((pseudo.shape[0],), jnp.int32)
        for d in range(3):
            b = b * (frac[:, d] if bits[d] else (1.0 - frac[:, d]))
            widx = widx * K + (i0[:, d] + bits[d])
        bs.append(b)
        ws.append(widx)
    return jnp.stack(bs, axis=1), jnp.stack(ws, axis=1)


def _spline_conv(x, src, dst, basis, widx, p):
    e = src.shape[0]
    w = p["w"]
    x_src = x[src]
    w_eff = jnp.zeros((e, w.shape[1], w.shape[2]), x.dtype)
    for s in range(8):
        w_eff = w_eff + basis[:, s, None, None] * w[widx[:, s]]
    msg = jnp.einsum('ei,eio->eo', x_src, w_eff)
    agg = jax.ops.segment_sum(msg, dst, num_segments=N_NODES)
    deg = jax.ops.segment_sum(jnp.ones((e,), x.dtype), dst, num_segments=N_NODES)
    agg = agg / jnp.maximum(deg, 1.0)[:, None]
    return agg + x @ p["root"] + p["b"]


def _phys_attn(x, p):
    b, n, c = x.shape
    h, dh = 8, 4
    fx = _linear(x, p["fx"]).reshape(b, n, h, dh).transpose(0, 2, 1, 3)
    xm = _linear(x, p["x"]).reshape(b, n, h, dh).transpose(0, 2, 1, 3)
    sw = jax.nn.softmax(_linear(xm, p["slice"]) / p["temp"], axis=-1)
    snorm = sw.sum(axis=2)
    st = jnp.einsum('bhnc,bhng->bhgc', fx, sw)
    st = st / (snorm + 1e-5)[:, :, :, None]
    q = st @ p["q"].T
    k = st @ p["k"].T
    v = st @ p["v"].T
    attn = jax.nn.softmax((q @ jnp.swapaxes(k, -1, -2)) * (dh ** -0.5), axis=-1)
    out = jnp.einsum('bhgc,bhng->bhnc', attn @ v, sw)
    out = out.transpose(0, 2, 1, 3).reshape(b, n, h * dh)
    return _linear(out, p["out"])


def kernel(x, edge_index, edge_attr, pos, params):
    src = edge_index[0]
    dst = edge_index[1]
    ef = _edge_feat_pallas(pos, src)
    e = _linear(jax.nn.gelu(_linear(ef, params["ee0"]), approximate=False), params["ee1"])
    basis, widx = _spline_basis(edge_attr)
    h = x
    for i in range(3):
        h = _bn(jax.nn.elu(_spline_conv(h, src, dst, basis, widx, params["convs"][i])), params["bns"][i])
    xb = (h + _linear(e, params["ep1"]))[None]
    xb = _phys_attn(_ln(xb, params["ln1"]), params["pa1"]) + xb
    xb = _mlp(xb, params["mlp1"]) + xb
    h = xb[0]
    for i in range(3, 6):
        h = _bn(jax.nn.elu(_spline_conv(h, src, dst, basis, widx, params["convs"][i])), params["bns"][i])
    xb = (h + _linear(e, params["ep2"]))[None]
    xb = _phys_attn(_ln(xb, params["ln2"]), params["pa2"]) + xb
    xb = _mlp(xb, params["mlp2"]) + xb
    h = xb[0]
    for i in range(6, 11):
        h = _bn(jax.nn.elu(_spline_conv(h, src, dst, basis, widx, params["convs"][i])), params["bns"][i])
    return _spline_conv(h, src, dst, basis, widx, params["convs"][11])
